# Initial kernel scaffold; baseline (speedup 1.0000x reference)
import jax
import jax.numpy as jnp
from jax.experimental import pallas as pl


def kernel(x, token):
    raise NotImplementedError("write your pallas kernel here")


# speedup vs baseline: 1.0376x; 1.0376x over previous
import functools

import jax
import jax.numpy as jnp
from jax.experimental import pallas as pl


def _body(x_ref, tok_ref, o_ref, *, blk, n):
    j = pl.program_id(1)
    rows = j * blk + jax.lax.broadcasted_iota(jnp.int32, (1, blk, 1), 1)
    keep = (rows >= n // 2 - 1) & (rows <= n - 2)
    o_ref[...] = jnp.where(keep, x_ref[...], tok_ref[...][None, None, :])


def kernel(x, token):
    B, N, D = x.shape
    BLK = 512
    nblk = N // BLK
    pin = (N // 2 - 1) // BLK

    return pl.pallas_call(
        functools.partial(_body, blk=BLK, n=N),
        grid=(B, nblk),
        in_specs=[
            pl.BlockSpec(
                (1, BLK, D),
                lambda b, j: (b, jnp.maximum(j, pin), 0),
            ),
            pl.BlockSpec((D,), lambda b, j: (0,)),
        ],
        out_specs=pl.BlockSpec((1, BLK, D), lambda b, j: (b, j, 0)),
        out_shape=jax.ShapeDtypeStruct((B, N, D), x.dtype),
    )(x, token)
